# baseline (speedup 1.0000x reference)
import functools

import jax
import jax.numpy as jnp
from jax import lax
from jax.experimental import pallas as pl
from jax.experimental.pallas import tpu as pltpu
from jax.experimental.pallas import tpu_sc as plsc

B, S, HID, MAXN = 32, 512, 1024, 512
HALF = HID // 2
L = 16
NC, NS = 2, 16
NW = NC * NS
C = 16
NCH = B
JV = HID // L
JH = HALF // L
EPS = 1e-12


def _lane_sum(v):
    idx0 = jnp.arange(L, dtype=jnp.int32)
    dnums = lax.GatherDimensionNumbers(
        offset_dims=(), collapsed_slice_dims=(0,), start_index_map=(0,))
    for k in (8, 4, 2, 1):
        rot = lax.gather(v, ((idx0 + k) % L)[:, None], dnums, (1,),
                         mode=lax.GatherScatterMode.PROMISE_IN_BOUNDS)
        v = v + rot
    return v


def _rsqrt_vec(x):
    i = lax.bitcast_convert_type(x, jnp.int32)
    i = jnp.full((L,), 0x5F3759DF, jnp.int32) - (i >> 1)
    y = lax.bitcast_convert_type(i, jnp.float32)
    for _ in range(3):
        y = y * (1.5 - 0.5 * x * y * y)
    return y


_mesh = plsc.VectorSubcoreMesh(core_axis_name="c", subcore_axis_name="s")


@functools.partial(
    pl.kernel,
    out_type=jax.ShapeDtypeStruct((B * 2 * S, HALF), jnp.float32),
    mesh=_mesh,
    scratch_types=[
        pltpu.VMEM((2 * S,), jnp.int32),
        pltpu.VMEM((2 * C, HALF), jnp.float32),
        pltpu.VMEM((2 * C, HALF), jnp.float32),
        pltpu.VMEM((2 * C, HALF), jnp.float32),
        pltpu.VMEM((2 * C, HALF), jnp.float32),
        pltpu.VMEM((2 * C, HALF), jnp.float32),
        pltpu.VMEM((HID,), jnp.float32),
        pltpu.VMEM((HID,), jnp.float32),
        pltpu.VMEM((C * L,), jnp.float32),
        pltpu.VMEM((C * L,), jnp.float32),
        pltpu.VMEM((C * L,), jnp.float32),
        pltpu.VMEM((C * L,), jnp.float32),
        pltpu.SemaphoreType.DMA,
        pltpu.SemaphoreType.DMA,
        pltpu.SemaphoreType.DMA,
        pltpu.SemaphoreType.DMA,
    ],
)
def _sc_kernel(comb_hbm, pos2_hbm, idx_hbm, gam_hbm, bet_hbm, out_hbm,
               idx_v, g0, g1, p_res, o0, o1, gam_v, bet_v, m_v, rs_v,
               ts_v, tq_v, gsem0, gsem1, osem0, osem1):
    w = lax.axis_index("s") * NC + lax.axis_index("c")
    base = pl.multiple_of(w * (2 * S), 2 * S)
    stripe = pl.multiple_of(w * (2 * C), 2 * C)
    pltpu.sync_copy(idx_hbm.at[pl.ds(base, 2 * S)], idx_v)
    pltpu.sync_copy(pos2_hbm.at[pl.ds(stripe, 2 * C)], p_res)
    pltpu.sync_copy(gam_hbm, gam_v)
    pltpu.sync_copy(bet_hbm, bet_v)

    def issue_in(ci, g_buf, gsem):
        r0 = pl.multiple_of(ci * (2 * C), 2 * C)
        pltpu.async_copy(comb_hbm.at[idx_v.at[pl.ds(r0, 2 * C)]], g_buf, gsem)

    def wait_in(g_buf, gsem):
        pltpu.make_async_copy(comb_hbm.at[pl.ds(0, 2 * C)], g_buf,
                              gsem).wait()

    def issue_out(ci, o_buf, osem):
        r0 = pl.multiple_of(2 * (ci * S + w * C), 2 * C)
        pltpu.async_copy(o_buf, out_hbm.at[pl.ds(r0, 2 * C)], osem)

    def wait_out(o_buf, osem):
        pltpu.make_async_copy(o_buf, out_hbm.at[pl.ds(0, 2 * C)],
                              osem).wait()

    def pass1(g_buf):
        def one_row(r):
            a_s = [jnp.zeros((L,), jnp.float32) for _ in range(4)]
            a_q = [jnp.zeros((L,), jnp.float32) for _ in range(4)]
            for j in range(JV):
                row = 2 * r + (j // JH)
                col = (j % JH) * L
                v = g_buf[row, pl.ds(col, L)] + p_res[row, pl.ds(col, L)]
                g_buf[row, pl.ds(col, L)] = v
                k = j % 4
                a_s[k] = a_s[k] + v
                a_q[k] = a_q[k] + v * v
            ts_v[pl.ds(r * L, L)] = (a_s[0] + a_s[1]) + (a_s[2] + a_s[3])
            tq_v[pl.ds(r * L, L)] = (a_q[0] + a_q[1]) + (a_q[2] + a_q[3])

        def row_body(q, c1):
            one_row(2 * q)
            one_row(2 * q + 1)
            return c1

        lax.fori_loop(0, C // 2, row_body, 0)

        lane = jnp.arange(L, dtype=jnp.int32)
        dnums = lax.GatherDimensionNumbers(
            offset_dims=(), collapsed_slice_dims=(0,), start_index_map=(0,))

        def perm(v, idx):
            return lax.gather(v, idx[:, None], dnums, (1,),
                              mode=lax.GatherScatterMode.PROMISE_IN_BOUNDS)

        def rowsums(ref):
            vecs = [ref[pl.ds(r * L, L)] for r in range(C)]
            k = 1
            while len(vecs) > 1:
                mask = (lane & k) != 0
                xidx = lane ^ k
                nxt = []
                for i in range(0, len(vecs), 2):
                    ah = vecs[i] + perm(vecs[i], xidx)
                    bh = vecs[i + 1] + perm(vecs[i + 1], xidx)
                    nxt.append(jnp.where(mask, bh, ah))
                vecs = nxt
                k *= 2
            return vecs[0]

        mean = rowsums(ts_v) * (1.0 / HID)
        var = rowsums(tq_v) * (1.0 / HID) - mean * mean
        rstd = _rsqrt_vec(var + EPS)
        for t in range(C):
            sel = jnp.full((L,), t, jnp.int32)
            m_v[pl.ds(t * L, L)] = perm(mean, sel)
            rs_v[pl.ds(t * L, L)] = perm(rstd, sel)

    def pass2(g_buf, o_buf):
        jper = 16
        for jb in range(JV // jper):
            gs = tuple(gam_v[pl.ds((jb * jper + t) * L, L)]
                       for t in range(jper))
            bs = tuple(bet_v[pl.ds((jb * jper + t) * L, L)]
                       for t in range(jper))

            def one_row2(r, cgs, cbs, jb):
                m = m_v[pl.ds(r * L, L)]
                rs = rs_v[pl.ds(r * L, L)]
                for t in range(jper):
                    j = jb * jper + t
                    row = 2 * r + (j // JH)
                    col = (j % JH) * L
                    e = g_buf[row, pl.ds(col, L)]
                    o_buf[row, pl.ds(col, L)] = (e - m) * rs * cgs[t] + cbs[t]

            def row2(q, carry, jb=jb):
                cgs, cbs = carry
                one_row2(2 * q, cgs, cbs, jb)
                one_row2(2 * q + 1, cgs, cbs, jb)
                return carry

            lax.fori_loop(0, C // 2, row2, (gs, bs))

    issue_in(0, g0, gsem0)

    def body(t, carry):
        i0 = 2 * t
        issue_in(i0 + 1, g1, gsem1)
        wait_in(g0, gsem0)
        pass1(g0)
        pl.when(t >= 1)(lambda: wait_out(o0, osem0))
        pass2(g0, o0)
        issue_out(i0, o0, osem0)
        pl.when(t < NCH // 2 - 1)(lambda: issue_in(i0 + 2, g0, gsem0))
        wait_in(g1, gsem1)
        pass1(g1)
        pl.when(t >= 1)(lambda: wait_out(o1, osem1))
        pass2(g1, o1)
        issue_out(i0 + 1, o1, osem1)
        return carry

    lax.fori_loop(0, NCH // 2, body, 0)
    wait_out(o0, osem0)
    wait_out(o1, osem1)


def kernel(top_vecs, tok_struct_vec, sent_struct_vec, pos_table, a_table,
           b_table, ln_gamma, ln_beta):
    del top_vecs, tok_struct_vec
    pa = sent_struct_vec[:, :, 0].astype(jnp.int32)
    sb = sent_struct_vec[:, :, 1].astype(jnp.int32) + MAXN
    idx = jnp.stack([pa, sb], axis=-1)
    idx = idx.reshape(B, NW, C, 2).transpose(1, 0, 2, 3).reshape(B * 2 * S)
    comb = jnp.concatenate([a_table, b_table], axis=0)
    pos2 = pos_table.reshape(2 * S, HALF)
    out = _sc_kernel(comb, pos2, idx, ln_gamma, ln_beta)
    return out.reshape(B, S, HID)


# speedup vs baseline: 1.3068x; 1.3068x over previous
import functools

import jax
import jax.numpy as jnp
from jax import lax
from jax.experimental import pallas as pl
from jax.experimental.pallas import tpu as pltpu
from jax.experimental.pallas import tpu_sc as plsc

B, S, HID, MAXN = 32, 512, 1024, 512
HALF = HID // 2
L = 16
NC, NS = 2, 16
NW = NC * NS
C = 16
NCH = B
JV = HID // L
JH = HALF // L
EPS = 1e-12


def _lane_sum(v):
    idx0 = jnp.arange(L, dtype=jnp.int32)
    dnums = lax.GatherDimensionNumbers(
        offset_dims=(), collapsed_slice_dims=(0,), start_index_map=(0,))
    for k in (8, 4, 2, 1):
        rot = lax.gather(v, ((idx0 + k) % L)[:, None], dnums, (1,),
                         mode=lax.GatherScatterMode.PROMISE_IN_BOUNDS)
        v = v + rot
    return v


def _rsqrt_vec(x):
    i = lax.bitcast_convert_type(x, jnp.int32)
    i = jnp.full((L,), 0x5F3759DF, jnp.int32) - (i >> 1)
    y = lax.bitcast_convert_type(i, jnp.float32)
    for _ in range(3):
        y = y * (1.5 - 0.5 * x * y * y)
    return y


_mesh = plsc.VectorSubcoreMesh(core_axis_name="c", subcore_axis_name="s")


@functools.partial(
    pl.kernel,
    out_type=jax.ShapeDtypeStruct((B * 2 * S, HALF), jnp.float32),
    mesh=_mesh,
    scratch_types=[
        pltpu.VMEM((2 * S,), jnp.int32),
        pltpu.VMEM((2 * C, HALF), jnp.float32),
        pltpu.VMEM((2 * C, HALF), jnp.float32),
        pltpu.VMEM((2 * C, HALF), jnp.float32),
        pltpu.VMEM((2 * C, HALF), jnp.float32),
        pltpu.VMEM((2 * C, HALF), jnp.float32),
        pltpu.VMEM((HID,), jnp.float32),
        pltpu.VMEM((HID,), jnp.float32),
        pltpu.VMEM((C * L,), jnp.float32),
        pltpu.VMEM((C * L,), jnp.float32),
        pltpu.VMEM((C * L,), jnp.float32),
        pltpu.VMEM((C * L,), jnp.float32),
        pltpu.SemaphoreType.DMA,
        pltpu.SemaphoreType.DMA,
        pltpu.SemaphoreType.DMA,
        pltpu.SemaphoreType.DMA,
    ],
)
def _sc_kernel(comb_hbm, pos2_hbm, idx_hbm, gam_hbm, bet_hbm, out_hbm,
               idx_v, g0, g1, p_res, o0, o1, gam_v, bet_v, m_v, rs_v,
               ts_v, tq_v, gsem0, gsem1, osem0, osem1):
    w = lax.axis_index("s") * NC + lax.axis_index("c")
    base = pl.multiple_of(w * (2 * S), 2 * S)
    stripe = pl.multiple_of(w * (2 * C), 2 * C)
    pltpu.sync_copy(idx_hbm.at[pl.ds(base, 2 * S)], idx_v)
    pltpu.sync_copy(pos2_hbm.at[pl.ds(stripe, 2 * C)], p_res)
    pltpu.sync_copy(gam_hbm, gam_v)
    pltpu.sync_copy(bet_hbm, bet_v)

    def issue_in(ci, g_buf, gsem):
        r0 = pl.multiple_of(ci * (2 * C), 2 * C)
        pltpu.async_copy(comb_hbm.at[idx_v.at[pl.ds(r0, 2 * C)]], g_buf, gsem)

    def wait_in(g_buf, gsem):
        pltpu.make_async_copy(comb_hbm.at[pl.ds(0, 2 * C)], g_buf,
                              gsem).wait()

    def issue_out(ci, o_buf, osem):
        r0 = pl.multiple_of(2 * (ci * S + w * C), 2 * C)
        pltpu.async_copy(o_buf, out_hbm.at[pl.ds(r0, 2 * C)], osem)

    def wait_out(o_buf, osem):
        pltpu.make_async_copy(o_buf, out_hbm.at[pl.ds(0, 2 * C)],
                              osem).wait()

    def pass1(g_buf):
        def row_body(r, c1):
            a_s = [jnp.zeros((L,), jnp.float32) for _ in range(4)]
            a_q = [jnp.zeros((L,), jnp.float32) for _ in range(4)]
            for j in range(JV):
                row = 2 * r + (j // JH)
                col = (j % JH) * L
                v = g_buf[row, pl.ds(col, L)] + p_res[row, pl.ds(col, L)]
                g_buf[row, pl.ds(col, L)] = v
                k = j % 4
                a_s[k] = a_s[k] + v
                a_q[k] = a_q[k] + v * v
            ts_v[pl.ds(r * L, L)] = (a_s[0] + a_s[1]) + (a_s[2] + a_s[3])
            tq_v[pl.ds(r * L, L)] = (a_q[0] + a_q[1]) + (a_q[2] + a_q[3])
            return c1

        lax.fori_loop(0, C, row_body, 0)

        lane = jnp.arange(L, dtype=jnp.int32)
        dnums = lax.GatherDimensionNumbers(
            offset_dims=(), collapsed_slice_dims=(0,), start_index_map=(0,))

        def perm(v, idx):
            return lax.gather(v, idx[:, None], dnums, (1,),
                              mode=lax.GatherScatterMode.PROMISE_IN_BOUNDS)

        def rowsums(ref):
            vecs = [ref[pl.ds(r * L, L)] for r in range(C)]
            k = 1
            while len(vecs) > 1:
                mask = (lane & k) != 0
                xidx = lane ^ k
                nxt = []
                for i in range(0, len(vecs), 2):
                    ah = vecs[i] + perm(vecs[i], xidx)
                    bh = vecs[i + 1] + perm(vecs[i + 1], xidx)
                    nxt.append(jnp.where(mask, bh, ah))
                vecs = nxt
                k *= 2
            return vecs[0]

        mean = rowsums(ts_v) * (1.0 / HID)
        var = rowsums(tq_v) * (1.0 / HID) - mean * mean
        rstd = _rsqrt_vec(var + EPS)
        for t in range(C):
            sel = jnp.full((L,), t, jnp.int32)
            m_v[pl.ds(t * L, L)] = perm(mean, sel)
            rs_v[pl.ds(t * L, L)] = perm(rstd, sel)

    def pass2(g_buf, o_buf):
        jper = 16
        for jb in range(JV // jper):
            gs = tuple(gam_v[pl.ds((jb * jper + t) * L, L)]
                       for t in range(jper))
            bs = tuple(bet_v[pl.ds((jb * jper + t) * L, L)]
                       for t in range(jper))

            def row2(r, carry, jb=jb):
                cgs, cbs = carry
                m = m_v[pl.ds(r * L, L)]
                rs = rs_v[pl.ds(r * L, L)]
                for t in range(jper):
                    j = jb * jper + t
                    row = 2 * r + (j // JH)
                    col = (j % JH) * L
                    e = g_buf[row, pl.ds(col, L)]
                    o_buf[row, pl.ds(col, L)] = (e - m) * rs * cgs[t] + cbs[t]
                return carry

            lax.fori_loop(0, C, row2, (gs, bs))

    issue_in(0, g0, gsem0)

    def body(t, carry):
        i0 = 2 * t
        issue_in(i0 + 1, g1, gsem1)
        wait_in(g0, gsem0)
        pass1(g0)
        pl.when(t >= 1)(lambda: wait_out(o0, osem0))
        pass2(g0, o0)
        issue_out(i0, o0, osem0)
        pl.when(t < NCH // 2 - 1)(lambda: issue_in(i0 + 2, g0, gsem0))
        wait_in(g1, gsem1)
        pass1(g1)
        pl.when(t >= 1)(lambda: wait_out(o1, osem1))
        pass2(g1, o1)
        issue_out(i0 + 1, o1, osem1)
        return carry

    lax.fori_loop(0, NCH // 2, body, 0)
    wait_out(o0, osem0)
    wait_out(o1, osem1)


def kernel(top_vecs, tok_struct_vec, sent_struct_vec, pos_table, a_table,
           b_table, ln_gamma, ln_beta):
    del top_vecs, tok_struct_vec
    pa = sent_struct_vec[:, :, 0].astype(jnp.int32)
    sb = sent_struct_vec[:, :, 1].astype(jnp.int32) + MAXN
    idx = jnp.stack([pa, sb], axis=-1)
    idx = idx.reshape(B, NW, C, 2).transpose(1, 0, 2, 3).reshape(B * 2 * S)
    comb = jnp.concatenate([a_table, b_table], axis=0)
    pos2 = pos_table.reshape(2 * S, HALF)
    out = _sc_kernel(comb, pos2, idx, ln_gamma, ln_beta)
    return out.reshape(B, S, HID)
